# Initial kernel scaffold; baseline (speedup 1.0000x reference)
import jax
import jax.numpy as jnp
from jax.experimental import pallas as pl


def kernel(x, token_table, pos_table):
    raise NotImplementedError("write your pallas kernel here")


# speedup vs baseline: 3.6839x; 3.6839x over previous
import functools

import jax
import jax.numpy as jnp
from jax import lax
from jax.experimental import pallas as pl
from jax.experimental.pallas import tpu as pltpu
from jax.experimental.pallas import tpu_sc as plsc

D = 128
L = 200
B = 1024
NC = 2
NS = 16
NW = NC * NS
ROWS_PER_W = B // NW
CA = 128
CB = L - CA

_mesh = plsc.VectorSubcoreMesh(core_axis_name="c", subcore_axis_name="s")


@functools.partial(
    pl.kernel,
    mesh=_mesh,
    out_type=jax.ShapeDtypeStruct((B * L, D), jnp.float32),
    scratch_types=[
        pltpu.VMEM((CA,), jnp.int32),
        pltpu.VMEM((CB,), jnp.int32),
        pltpu.VMEM((CA, D), jnp.float32),
        pltpu.VMEM((CB, D), jnp.float32),
        pltpu.VMEM((CA, D), jnp.float32),
        pltpu.VMEM((CB, D), jnp.float32),
        pltpu.SemaphoreType.DMA,
        pltpu.SemaphoreType.DMA,
    ],
)
def _tok_pos_embed(x_hbm, tok_hbm, pos_hbm, out_hbm,
                   idx_a, idx_b, buf_a, buf_b, pos_a, pos_b, sem_a, sem_b):
    wid = lax.axis_index("s") * NC + lax.axis_index("c")

    pltpu.sync_copy(pos_hbm.at[pl.ds(0, CA)], pos_a)
    pltpu.sync_copy(pos_hbm.at[pl.ds(CA, CB)], pos_b)

    def row_body(r, carry):
        base = (wid * ROWS_PER_W + r) * L
        pltpu.sync_copy(x_hbm.at[pl.ds(base, CA)], idx_a)
        pltpu.sync_copy(x_hbm.at[pl.ds(base + CA, CB)], idx_b)
        ga = pltpu.async_copy(tok_hbm.at[idx_a], buf_a, sem_a)
        gb = pltpu.async_copy(tok_hbm.at[idx_b], buf_b, sem_b)
        ga.wait()
        gb.wait()

        def add_a(i, c):
            for j in range(D // 16):
                plsc.addupdate(buf_a.at[i, pl.ds(j * 16, 16)],
                               pos_a[i, pl.ds(j * 16, 16)])
            return c

        def add_b(i, c):
            for j in range(D // 16):
                plsc.addupdate(buf_b.at[i, pl.ds(j * 16, 16)],
                               pos_b[i, pl.ds(j * 16, 16)])
            return c

        lax.fori_loop(0, CA, add_a, 0)
        lax.fori_loop(0, CB, add_b, 0)

        pltpu.sync_copy(buf_a, out_hbm.at[pl.ds(base, CA)])
        pltpu.sync_copy(buf_b, out_hbm.at[pl.ds(base + CA, CB)])
        return carry

    lax.fori_loop(0, ROWS_PER_W, row_body, 0)


def kernel(x, token_table, pos_table):
    x_flat = x.reshape(-1).astype(jnp.int32)
    out = _tok_pos_embed(x_flat, token_table, pos_table)
    return out.reshape(B, L, D)
